# Initial kernel scaffold; baseline (speedup 1.0000x reference)
import jax
import jax.numpy as jnp
from jax.experimental import pallas as pl


def kernel(x, seq_encoding, person_encoding, num_people):
    raise NotImplementedError("write your pallas kernel here")


# speedup vs baseline: 1.6665x; 1.6665x over previous
import functools

import jax
import jax.numpy as jnp
from jax.experimental import pallas as pl
from jax.experimental.pallas import tpu as pltpu


def _enc_add_kernel(np_ref, seq_ref, per_ref, x_ref, o_ref, *, half):
    sq = seq_ref[...]
    snorm = jnp.sqrt(jnp.sum(sq * sq, axis=-1, keepdims=True))
    sscale = jnp.where(snorm > 1.0, 1.0 / (snorm + 1e-7), 1.0)
    sq_half = (sq * sscale)[:, :half]

    pt = per_ref[...]
    num_people = np_ref[0]
    p_cap = pt.shape[0]
    last = per_ref[pl.ds(jnp.minimum(num_people - 1, p_cap - 1), 1), :]
    pidx = jax.lax.broadcasted_iota(jnp.int32, (p_cap, 1), 0)
    psel = jnp.where(pidx < num_people, pt, last)
    pnorm = jnp.sqrt(jnp.sum(psel * psel, axis=-1, keepdims=True))
    pscale = jnp.where(pnorm > 1.0, 1.0 / (pnorm + 1e-7), 1.0)
    pt_half = (psel * pscale)[:, :half]

    xb = x_ref[0]
    bs, p, _ = xb.shape
    enc = jnp.concatenate(
        [
            jnp.broadcast_to(sq_half[:, None, :], (bs, p, half)),
            jnp.broadcast_to(pt_half[None, :, :], (bs, p, half)),
        ],
        axis=-1,
    )
    o_ref[0] = xb + enc


def kernel(x, seq_encoding, person_encoding, num_people):
    b, s, p, d = x.shape
    half = d // 2
    bs = next((c for c in (40, 48, 32, 24, 16, 8) if s % c == 0), s)
    seq_s = seq_encoding[:s]
    per_p = person_encoding[:p]
    np_arr = jnp.asarray(num_people, jnp.int32).reshape((1,))

    grid = (s // bs, b)
    return pl.pallas_call(
        functools.partial(_enc_add_kernel, half=half),
        grid=grid,
        in_specs=[
            pl.BlockSpec(memory_space=pltpu.SMEM),
            pl.BlockSpec((bs, d), lambda i, j: (i, 0)),
            pl.BlockSpec((p, d), lambda i, j: (0, 0)),
            pl.BlockSpec((1, bs, p, d), lambda i, j: (j, i, 0, 0)),
        ],
        out_specs=pl.BlockSpec((1, bs, p, d), lambda i, j: (j, i, 0, 0)),
        out_shape=jax.ShapeDtypeStruct((b, s, p, d), x.dtype),
    )(np_arr, seq_s, per_p, x)


# speedup vs baseline: 1.8305x; 1.0984x over previous
import functools

import jax
import jax.numpy as jnp
from jax.experimental import pallas as pl
from jax.experimental.pallas import tpu as pltpu


def _enc_add_kernel(np_ref, seq_ref, per_ref, x_ref, o_ref, enc_ref, *, half):
    @pl.when(pl.program_id(1) == 0)
    def _build_enc():
        sq = seq_ref[...]
        snorm = jnp.sqrt(jnp.sum(sq * sq, axis=-1, keepdims=True))
        sscale = jnp.where(snorm > 1.0, 1.0 / (snorm + 1e-7), 1.0)
        sq_half = (sq * sscale)[:, :half]

        pt = per_ref[...]
        num_people = np_ref[0]
        p_cap = pt.shape[0]
        last = per_ref[pl.ds(jnp.minimum(num_people - 1, p_cap - 1), 1), :]
        pidx = jax.lax.broadcasted_iota(jnp.int32, (p_cap, 1), 0)
        psel = jnp.where(pidx < num_people, pt, last)
        pnorm = jnp.sqrt(jnp.sum(psel * psel, axis=-1, keepdims=True))
        pscale = jnp.where(pnorm > 1.0, 1.0 / (pnorm + 1e-7), 1.0)
        pt_half = (psel * pscale)[:, :half]

        bs = sq.shape[0]
        enc_ref[...] = jnp.concatenate(
            [
                jnp.broadcast_to(sq_half[:, None, :], (bs, p_cap, half)),
                jnp.broadcast_to(pt_half[None, :, :], (bs, p_cap, half)),
            ],
            axis=-1,
        )

    o_ref[0] = x_ref[0] + enc_ref[...]


def kernel(x, seq_encoding, person_encoding, num_people):
    b, s, p, d = x.shape
    half = d // 2
    bs = next((c for c in (40, 48, 32, 24, 16, 8) if s % c == 0), s)
    seq_s = seq_encoding[:s]
    per_p = person_encoding[:p]
    np_arr = jnp.asarray(num_people, jnp.int32).reshape((1,))

    grid = (s // bs, b)
    return pl.pallas_call(
        functools.partial(_enc_add_kernel, half=half),
        grid=grid,
        in_specs=[
            pl.BlockSpec(memory_space=pltpu.SMEM),
            pl.BlockSpec((bs, d), lambda i, j: (i, 0)),
            pl.BlockSpec((p, d), lambda i, j: (0, 0)),
            pl.BlockSpec((1, bs, p, d), lambda i, j: (j, i, 0, 0)),
        ],
        out_specs=pl.BlockSpec((1, bs, p, d), lambda i, j: (j, i, 0, 0)),
        out_shape=jax.ShapeDtypeStruct((b, s, p, d), x.dtype),
        scratch_shapes=[pltpu.VMEM((bs, p, d), x.dtype)],
    )(np_arr, seq_s, per_p, x)


# speedup vs baseline: 3.4290x; 1.8733x over previous
import functools

import jax
import jax.numpy as jnp
from jax.experimental import pallas as pl
from jax.experimental.pallas import tpu as pltpu


def _enc_add_kernel(np_ref, seq_ref, per_ref, x_ref, o_ref, enc_ref, *, half):
    @pl.when(pl.program_id(1) == 0)
    def _build_enc():
        sq = seq_ref[...]
        snorm = jnp.sqrt(jnp.sum(sq * sq, axis=-1, keepdims=True))
        sscale = jnp.where(snorm > 1.0, 1.0 / (snorm + 1e-7), 1.0)
        sq_half = (sq * sscale)[:, :half]

        pt = per_ref[...]
        num_people = np_ref[0]
        p_cap = pt.shape[0]
        last = per_ref[pl.ds(jnp.minimum(num_people - 1, p_cap - 1), 1), :]
        pidx = jax.lax.broadcasted_iota(jnp.int32, (p_cap, 1), 0)
        psel = jnp.where(pidx < num_people, pt, last)
        pnorm = jnp.sqrt(jnp.sum(psel * psel, axis=-1, keepdims=True))
        pscale = jnp.where(pnorm > 1.0, 1.0 / (pnorm + 1e-7), 1.0)
        pt_half = (psel * pscale)[:, :half]

        bs = sq.shape[0]
        enc_ref[...] = jnp.concatenate(
            [
                jnp.broadcast_to(sq_half[:, None, :], (bs, p_cap, half)),
                jnp.broadcast_to(pt_half[None, :, :], (bs, p_cap, half)),
            ],
            axis=-1,
        )

    o_ref[0] = x_ref[0] + enc_ref[...]


def kernel(x, seq_encoding, person_encoding, num_people):
    b, s, p, d = x.shape
    half = d // 2
    bs = s
    seq_s = seq_encoding[:s]
    per_p = person_encoding[:p]
    np_arr = jnp.asarray(num_people, jnp.int32).reshape((1,))

    grid = (s // bs, b)
    return pl.pallas_call(
        functools.partial(_enc_add_kernel, half=half),
        grid=grid,
        in_specs=[
            pl.BlockSpec(memory_space=pltpu.SMEM),
            pl.BlockSpec((bs, d), lambda i, j: (i, 0)),
            pl.BlockSpec((p, d), lambda i, j: (0, 0)),
            pl.BlockSpec((1, bs, p, d), lambda i, j: (j, i, 0, 0)),
        ],
        out_specs=pl.BlockSpec((1, bs, p, d), lambda i, j: (j, i, 0, 0)),
        out_shape=jax.ShapeDtypeStruct((b, s, p, d), x.dtype),
        scratch_shapes=[pltpu.VMEM((bs, p, d), x.dtype)],
    )(np_arr, seq_s, per_p, x)


# speedup vs baseline: 3.5244x; 1.0278x over previous
import functools

import jax
import jax.numpy as jnp
from jax.experimental import pallas as pl
from jax.experimental.pallas import tpu as pltpu


def _enc_add_kernel(np_ref, seq_ref, per_ref, x_ref, o_ref, enc_ref, *, half):
    @pl.when(pl.program_id(1) == 0)
    def _build_enc():
        sq = seq_ref[...]
        snorm = jnp.sqrt(jnp.sum(sq * sq, axis=-1, keepdims=True))
        sscale = jnp.where(snorm > 1.0, 1.0 / (snorm + 1e-7), 1.0)
        sq_half = (sq * sscale)[:, :half]

        pt = per_ref[...]
        num_people = np_ref[0]
        p_cap = pt.shape[0]
        last = per_ref[pl.ds(jnp.minimum(num_people - 1, p_cap - 1), 1), :]
        pidx = jax.lax.broadcasted_iota(jnp.int32, (p_cap, 1), 0)
        psel = jnp.where(pidx < num_people, pt, last)
        pnorm = jnp.sqrt(jnp.sum(psel * psel, axis=-1, keepdims=True))
        pscale = jnp.where(pnorm > 1.0, 1.0 / (pnorm + 1e-7), 1.0)
        pt_half = (psel * pscale)[:, :half]

        bs = sq.shape[0]
        enc_ref[...] = jnp.concatenate(
            [
                jnp.broadcast_to(sq_half[:, None, :], (bs, p_cap, half)),
                jnp.broadcast_to(pt_half[None, :, :], (bs, p_cap, half)),
            ],
            axis=-1,
        )

    o_ref[...] = x_ref[...] + enc_ref[...]


def kernel(x, seq_encoding, person_encoding, num_people):
    b, s, p, d = x.shape
    half = d // 2
    bs = s
    bb = next((c for c in (2, 4) if b % c == 0), 1)
    seq_s = seq_encoding[:s]
    per_p = person_encoding[:p]
    np_arr = jnp.asarray(num_people, jnp.int32).reshape((1,))

    grid = (s // bs, b // bb)
    return pl.pallas_call(
        functools.partial(_enc_add_kernel, half=half),
        grid=grid,
        in_specs=[
            pl.BlockSpec(memory_space=pltpu.SMEM),
            pl.BlockSpec((bs, d), lambda i, j: (i, 0)),
            pl.BlockSpec((p, d), lambda i, j: (0, 0)),
            pl.BlockSpec((bb, bs, p, d), lambda i, j: (j, i, 0, 0)),
        ],
        out_specs=pl.BlockSpec((bb, bs, p, d), lambda i, j: (j, i, 0, 0)),
        out_shape=jax.ShapeDtypeStruct((b, s, p, d), x.dtype),
        scratch_shapes=[pltpu.VMEM((bs, p, d), x.dtype)],
    )(np_arr, seq_s, per_p, x)


# speedup vs baseline: 3.5772x; 1.0150x over previous
import functools

import jax
import jax.numpy as jnp
from jax.experimental import pallas as pl
from jax.experimental.pallas import tpu as pltpu


def _enc_add_kernel(np_ref, seq_ref, per_ref, x_ref, o_ref, enc_ref, *, half):
    @pl.when(pl.program_id(1) == 0)
    def _build_enc():
        sq = seq_ref[...]
        snorm = jnp.sqrt(jnp.sum(sq * sq, axis=-1, keepdims=True))
        sscale = jnp.where(snorm > 1.0, 1.0 / (snorm + 1e-7), 1.0)
        sq_half = (sq * sscale)[:, :half]

        pt = per_ref[...]
        num_people = np_ref[0]
        p_cap = pt.shape[0]
        last = per_ref[pl.ds(jnp.minimum(num_people - 1, p_cap - 1), 1), :]
        pidx = jax.lax.broadcasted_iota(jnp.int32, (p_cap, 1), 0)
        psel = jnp.where(pidx < num_people, pt, last)
        pnorm = jnp.sqrt(jnp.sum(psel * psel, axis=-1, keepdims=True))
        pscale = jnp.where(pnorm > 1.0, 1.0 / (pnorm + 1e-7), 1.0)
        pt_half = (psel * pscale)[:, :half]

        bs = sq.shape[0]
        enc_ref[...] = jnp.concatenate(
            [
                jnp.broadcast_to(sq_half[:, None, :], (bs, p_cap, half)),
                jnp.broadcast_to(pt_half[None, :, :], (bs, p_cap, half)),
            ],
            axis=-1,
        )

    o_ref[...] = x_ref[...] + enc_ref[...]


def kernel(x, seq_encoding, person_encoding, num_people):
    b, s, p, d = x.shape
    half = d // 2
    bs = s
    bb = next((c for c in (4, 2) if b % c == 0), 1)
    seq_s = seq_encoding[:s]
    per_p = person_encoding[:p]
    np_arr = jnp.asarray(num_people, jnp.int32).reshape((1,))

    grid = (s // bs, b // bb)
    return pl.pallas_call(
        functools.partial(_enc_add_kernel, half=half),
        grid=grid,
        in_specs=[
            pl.BlockSpec(memory_space=pltpu.SMEM),
            pl.BlockSpec((bs, d), lambda i, j: (i, 0)),
            pl.BlockSpec((p, d), lambda i, j: (0, 0)),
            pl.BlockSpec((bb, bs, p, d), lambda i, j: (j, i, 0, 0)),
        ],
        out_specs=pl.BlockSpec((bb, bs, p, d), lambda i, j: (j, i, 0, 0)),
        out_shape=jax.ShapeDtypeStruct((b, s, p, d), x.dtype),
        scratch_shapes=[pltpu.VMEM((bs, p, d), x.dtype)],
    )(np_arr, seq_s, per_p, x)
